# baseline (speedup 1.0000x reference)
import jax
import jax.numpy as jnp
from jax import lax
from jax.experimental import pallas as pl
from jax.experimental.pallas import tpu as pltpu
from jax.experimental.pallas import tpu_sc as plsc

_NC, _NS = 2, 16
_NW = _NC * _NS
_D = 128
_B = 4096 * 200
_IPW = _B // _NW

_NST = 32
_SL = 100000 // _NST
_CAP = 896
_CH = 64
_NCHB = _CAP // _CH
_XCH = 800


def _body(x_hbm, emb_hbm, out_hbm,
          xbuf, valb, posb, curs, ilist, ob0, ob1, slice_spm,
          gsem, s0, s1):
    obufs = (ob0, ob1)
    ssem = (s0, s1)
    cid = lax.axis_index("c")
    sid = lax.axis_index("s")
    wid = sid * _NC + cid
    w_base = wid * _IPW
    lane = lax.iota(jnp.int32, 16)

    curs[pl.ds(0, 16)] = jnp.zeros((16,), jnp.int32)
    curs[pl.ds(16, 16)] = jnp.zeros((16,), jnp.int32)

    def round_(c8, carry):
        pltpu.sync_copy(x_hbm.at[pl.ds(w_base + c8 * _XCH, _XCH)], xbuf)

        def binstep(i, carry2):
            v = xbuf[pl.ds(i * 16, 16)]
            pos = (w_base + c8 * _XCH + i * 16) + lane
            bkt = lax.div(v, _SL)
            rank, last = plsc.scan_count(bkt)
            cur = plsc.load_gather(curs, [bkt])
            slot = jnp.clip(cur + rank - 1, 0, _CAP - 1)
            dst = bkt * _CAP + slot
            plsc.store_scatter(valb, [dst], v)
            plsc.store_scatter(posb, [lax.div(dst, _CH), lax.rem(dst, _CH)],
                               pos)
            plsc.store_scatter(curs, [bkt], cur + rank, mask=last)
            return carry2

        lax.fori_loop(0, _XCH // 16, binstep, 0)
        return carry

    lax.fori_loop(0, _IPW // _XCH, round_, 0)

    def bucket_count(s):
        g = curs[pl.ds(lax.div(s, 16) * 16, 16)]
        return jnp.max(jnp.where(lane == lax.rem(s, 16), g, 0))

    def pad(s, carry):
        cnt = bucket_count(s)
        pad_end = lax.div(cnt + (_CH - 1), _CH) * _CH
        base = s * _CAP
        bsplat = jnp.zeros((16,), jnp.int32) + base
        fv = plsc.load_gather(valb, [bsplat])
        fp = plsc.load_gather(posb, [lax.div(bsplat, _CH),
                                     lax.rem(bsplat, _CH)])
        for k in range(4):
            offs = cnt + k * 16 + lane
            m = offs < pad_end
            dst = base + jnp.minimum(offs, _CAP - 1)
            plsc.store_scatter(valb, [dst], fv, mask=m)
            plsc.store_scatter(posb, [lax.div(dst, _CH), lax.rem(dst, _CH)],
                               fp, mask=m)
        return carry

    lax.fori_loop(0, _NST, pad, 0)

    def stage(s, carry):
        lo = (s * _SL) & ~7
        n = (((s + 1) * _SL + 7) & ~7) - lo
        plsc.subcore_barrier()
        row0 = pl.multiple_of(jnp.minimum(sid * 200, n - 200), 8)
        lo8 = pl.multiple_of(lo + row0, 8)
        pltpu.sync_copy(emb_hbm.at[pl.ds(lo8, 200)],
                        slice_spm.at[pl.ds(row0, 200)])
        plsc.subcore_barrier()

        cnt = bucket_count(s)
        nch = lax.div(cnt + (_CH - 1), _CH)

        def pair(cp, carry2):
            for kq in range(2):
                c = jnp.minimum(cp * 2 + kq, nch - 1)
                off = s * _CAP + c * _CH

                @pl.when(cp > 0)
                def _wait_prev(kq=kq):
                    pltpu.make_async_copy(obufs[kq],
                                          out_hbm.at[posb.at[0]],
                                          ssem[kq]).wait()

                for k in range(4):
                    lv = jnp.clip(valb[pl.ds(off + k * 16, 16)] - lo,
                                  0, 3135)
                    ilist[pl.ds(k * 16, 16)] = lv
                pltpu.async_copy(slice_spm.at[ilist], obufs[kq], gsem).wait()
                pltpu.async_copy(obufs[kq],
                                 out_hbm.at[posb.at[s * _NCHB + c]],
                                 ssem[kq])
            return carry2

        lax.fori_loop(0, lax.div(nch + 1, 2), pair, 0)

        @pl.when(nch > 0)
        def _drain():
            for kq in range(2):
                pltpu.make_async_copy(obufs[kq], out_hbm.at[posb.at[0]],
                                      ssem[kq]).wait()

        return carry

    lax.fori_loop(0, _NST, stage, 0)


def kernel(x, emb):
    xf = x.reshape(_B)
    mesh = plsc.VectorSubcoreMesh(core_axis_name="c", subcore_axis_name="s")
    out = pl.kernel(
        _body,
        out_type=jax.ShapeDtypeStruct((_B, _D), jnp.float32),
        mesh=mesh,
        compiler_params=pltpu.CompilerParams(needs_layout_passes=False),
        scratch_types=[
            pltpu.VMEM((_XCH,), jnp.int32),
            pltpu.VMEM((_NST * _CAP,), jnp.int32),
            pltpu.VMEM((_NST * _NCHB, _CH), jnp.int32),
            pltpu.VMEM((32,), jnp.int32),
            pltpu.VMEM((_CH,), jnp.int32),
            pltpu.VMEM((_CH, _D), jnp.float32),
            pltpu.VMEM((_CH, _D), jnp.float32),
            pltpu.VMEM_SHARED((3136, _D), jnp.float32),
            pltpu.SemaphoreType.DMA,
            pltpu.SemaphoreType.DMA,
            pltpu.SemaphoreType.DMA,
        ],
    )(xf, emb)
    return out.reshape(x.shape[0], x.shape[1], _D)


# speedup vs baseline: 1.6249x; 1.6249x over previous
import jax
import jax.numpy as jnp
from jax import lax
from jax.experimental import pallas as pl
from jax.experimental.pallas import tpu as pltpu
from jax.experimental.pallas import tpu_sc as plsc

_NC, _NS = 2, 16
_NW = _NC * _NS
_D = 128
_B = 4096 * 200
_ROWS = _B // _D
_RPW = _ROWS // _NW

_NB = 4
_AHEAD = 4


def _body(x_hbm, emb_hbm, out_hbm, idx_v, spm_all,
          b0, b1, b2, b3,
          g0, g1, g2, g3,
          a0, a1, w0, w1):
    bufs = (b0, b1, b2, b3)
    gs = (g0, g1, g2, g3)
    asem = (a0, a1)
    bsem = (w0, w1)
    wid = lax.axis_index("s") * _NC + lax.axis_index("c")
    spm = spm_all.at[lax.axis_index("s")]
    r0 = wid * _RPW
    pltpu.sync_copy(x_hbm.at[pl.ds(r0, _RPW)], idx_v)

    def wait_gather(t, b):
        pltpu.make_async_copy(emb_hbm.at[idx_v.at[t]], bufs[b], gs[b]).wait()

    def step(t, b, q, first):
        wait_gather(t, b)
        if not first:
            pltpu.make_async_copy(spm.at[q], out_hbm.at[pl.ds(r0 * _D, _D)],
                                  bsem[q]).wait()
        pltpu.async_copy(bufs[b], spm.at[q], asem[q]).wait()
        pltpu.async_copy(spm.at[q], out_hbm.at[pl.ds((r0 + t) * _D, _D)],
                         bsem[q])

    for t in range(_AHEAD):
        pltpu.async_copy(emb_hbm.at[idx_v.at[t]], bufs[t % _NB], gs[t % _NB])
    for t in range(_NB):
        step(t, t % _NB, t % 2, first=(t < 2))
        bn = (t + _AHEAD) % _NB
        pltpu.async_copy(emb_hbm.at[idx_v.at[t + _AHEAD]], bufs[bn], gs[bn])

    def outer(i, carry):
        for k in range(_NB):
            t = _NB + i * _NB + k
            step(t, k, k % 2, first=False)
            bn = (k + _AHEAD) % _NB
            tn = jnp.minimum(t + _AHEAD, _RPW - 1)
            pltpu.async_copy(emb_hbm.at[idx_v.at[tn]], bufs[bn], gs[bn])
        return carry

    lax.fori_loop(0, (_RPW - _NB) // _NB, outer, 0)

    for t in range(_RPW, _RPW + _AHEAD):
        wait_gather(0, t % _NB)
    for q in range(2):
        pltpu.make_async_copy(spm.at[q], out_hbm.at[pl.ds(r0 * _D, _D)],
                              bsem[q]).wait()


def kernel(x, emb):
    xf = x.reshape(_ROWS, _D)
    mesh = plsc.VectorSubcoreMesh(core_axis_name="c", subcore_axis_name="s")
    out = pl.kernel(
        _body,
        out_type=jax.ShapeDtypeStruct((_B, _D), jnp.float32),
        mesh=mesh,
        scratch_types=[
            pltpu.VMEM((_RPW, _D), jnp.int32),
            pltpu.VMEM_SHARED((_NS, 2, _D, _D), jnp.float32),
        ] + [pltpu.VMEM((_D, _D), jnp.float32)] * _NB
          + [pltpu.SemaphoreType.DMA] * (_NB + 4),
    )(xf, emb)
    return out.reshape(x.shape[0], x.shape[1], _D)
